# Initial kernel scaffold; baseline (speedup 1.0000x reference)
import jax
import jax.numpy as jnp
from jax.experimental import pallas as pl


def kernel(x, edge_index, pairs, W_enc, Omega, W_s_raw, W_fc1, W_fc2, W_fv, W_p1, W_p2):
    raise NotImplementedError("write your pallas kernel here")


# speedup vs baseline: 5.4942x; 5.4942x over previous
import functools

import jax
import jax.numpy as jnp
from jax import lax
from jax.experimental import pallas as pl
from jax.experimental.pallas import tpu as pltpu
from jax.experimental.pallas import tpu_sc as plsc

N = 10000
E = 320000
IN_DIM = 128
HID = 128
P = 8192
L = 20

NC = 2
NS = 16
NW = NC * NS
C = 128
NR = 10112
RPT = NR // NS
EPW_CH = 79
EP = NW * EPW_CH * C
GPW_CH = (2 * P) // (NW * C)
DUMP = N
DEGW = 128


@functools.cache
def _sc_kernels():
    mesh = plsc.VectorSubcoreMesh(core_axis_name="c", subcore_axis_name="s",
                                  num_cores=NC, num_subcores=NS)

    @functools.partial(
        pl.kernel,
        out_type=jax.ShapeDtypeStruct((NC, NR, HID), jnp.float32),
        mesh=mesh,
        scratch_types=[
            pltpu.VMEM((C,), jnp.int32),
            pltpu.VMEM((C,), jnp.int32),
            pltpu.VMEM((C, HID), jnp.float32),
            pltpu.VMEM_SHARED((NR, HID), jnp.float32),
            pltpu.SemaphoreType.DMA,
        ],
    )
    def edge_scatter(z_hbm, row_hbm, col_hbm, zeros_hbm, out_hbm,
                     rbuf, cbuf, gbuf, acc, sem):
        c = lax.axis_index("c")
        s = lax.axis_index("s")
        wid = s * NC + c

        @pl.when(s == 0)
        def _zero():
            pltpu.sync_copy(zeros_hbm, acc)

        plsc.subcore_barrier()

        def body(i, carry):
            base = (wid * EPW_CH + i) * C
            pltpu.sync_copy(row_hbm.at[pl.ds(base, C)], rbuf)
            pltpu.sync_copy(col_hbm.at[pl.ds(base, C)], cbuf)
            pltpu.async_copy(z_hbm.at[rbuf], gbuf, sem).wait()
            pltpu.sync_copy(gbuf, acc.at[cbuf], add=True)
            return carry

        lax.fori_loop(0, EPW_CH, body, 0)
        plsc.subcore_barrier()
        pltpu.sync_copy(acc.at[pl.ds(s * RPT, RPT)],
                        out_hbm.at[c, pl.ds(s * RPT, RPT)])

    @functools.partial(
        pl.kernel,
        out_type=jax.ShapeDtypeStruct((NC, NR, DEGW), jnp.float32),
        mesh=mesh,
        scratch_types=[
            pltpu.VMEM((C,), jnp.int32),
            pltpu.VMEM((C, DEGW), jnp.float32),
            pltpu.VMEM_SHARED((NR, DEGW), jnp.float32),
        ],
    )
    def deg_scatter(col_hbm, ones_hbm, zeros_hbm, out_hbm, cbuf, obuf, acc):
        c = lax.axis_index("c")
        s = lax.axis_index("s")
        wid = s * NC + c

        pltpu.sync_copy(ones_hbm, obuf)

        @pl.when(s == 0)
        def _zero():
            pltpu.sync_copy(zeros_hbm, acc)

        plsc.subcore_barrier()

        def body(i, carry):
            base = (wid * EPW_CH + i) * C
            pltpu.sync_copy(col_hbm.at[pl.ds(base, C)], cbuf)
            pltpu.sync_copy(obuf, acc.at[cbuf], add=True)
            return carry

        lax.fori_loop(0, EPW_CH, body, 0)
        plsc.subcore_barrier()
        pltpu.sync_copy(acc.at[pl.ds(s * RPT, RPT)],
                        out_hbm.at[c, pl.ds(s * RPT, RPT)])

    @functools.partial(
        pl.kernel,
        out_type=jax.ShapeDtypeStruct((2 * P, HID), jnp.float32),
        mesh=mesh,
        scratch_types=[
            pltpu.VMEM((C,), jnp.int32),
            pltpu.VMEM((C, HID), jnp.float32),
            pltpu.SemaphoreType.DMA,
        ],
    )
    def pair_gather(tab_hbm, idx_hbm, out_hbm, ibuf, gbuf, sem):
        c = lax.axis_index("c")
        s = lax.axis_index("s")
        wid = s * NC + c

        def body(i, carry):
            base = (wid * GPW_CH + i) * C
            pltpu.sync_copy(idx_hbm.at[pl.ds(base, C)], ibuf)
            pltpu.async_copy(tab_hbm.at[ibuf], gbuf, sem).wait()
            pltpu.sync_copy(gbuf, out_hbm.at[pl.ds(base, C)])
            return carry

        lax.fori_loop(0, GPW_CH, body, 0)

    return edge_scatter, deg_scatter, pair_gather


RB = 1000
NG = N // RB
PB = 1024
PG = P // PB

_CT1 = (((1,), (1,)), ((), ()))
_CT0 = (((1,), (0,)), ((), ()))


def _enc_body(x_ref, w_ref, h_ref):
    h_ref[...] = jnp.maximum(
        lax.dot_general(x_ref[...], w_ref[...], _CT1,
                        preferred_element_type=jnp.float32), 0.0)


_encode = pl.pallas_call(
    _enc_body,
    grid=(NG,),
    in_specs=[pl.BlockSpec((RB, IN_DIM), lambda i: (i, 0)),
              pl.BlockSpec((HID, IN_DIM), lambda i: (0, 0))],
    out_specs=pl.BlockSpec((RB, HID), lambda i: (i, 0)),
    out_shape=jax.ShapeDtypeStruct((N, HID), jnp.float32),
)


def _dis_body(dp_ref, dis_ref):
    deg = dp_ref[0, :, 0:1] + dp_ref[1, :, 0:1] + 1.0
    dis_ref[...] = lax.rsqrt(deg)


_dis_k = pl.pallas_call(
    _dis_body,
    in_specs=[pl.BlockSpec((NC, NR, DEGW), lambda: (0, 0, 0))],
    out_specs=pl.BlockSpec((NR, 1), lambda: (0, 0)),
    out_shape=jax.ShapeDtypeStruct((NR, 1), jnp.float32),
)


def _msg_body(h_ref, oa_ref, ws_ref, dis_ref, anti_ref, z_ref):
    h = h_ref[...]
    anti_ref[...] = -jnp.maximum(
        lax.dot_general(h, oa_ref[...], _CT0,
                        preferred_element_type=jnp.float32), 0.0)
    z_ref[...] = dis_ref[...] * lax.dot_general(
        h, ws_ref[...], _CT0, preferred_element_type=jnp.float32)


_msg = pl.pallas_call(
    _msg_body,
    grid=(NG,),
    in_specs=[pl.BlockSpec((RB, HID), lambda i: (i, 0)),
              pl.BlockSpec((HID, HID), lambda i: (0, 0)),
              pl.BlockSpec((HID, HID), lambda i: (0, 0)),
              pl.BlockSpec((RB, 1), lambda i: (i, 0))],
    out_specs=[pl.BlockSpec((RB, HID), lambda i: (i, 0)),
               pl.BlockSpec((RB, HID), lambda i: (i, 0))],
    out_shape=[jax.ShapeDtypeStruct((N, HID), jnp.float32),
               jax.ShapeDtypeStruct((N, HID), jnp.float32)],
)


def _upd_body(anti_ref, z_ref, p_ref, dis_ref, h_ref, hn_ref, dh_ref):
    sym = dis_ref[...] * (p_ref[0] + p_ref[1] + z_ref[...])
    dh = jnp.maximum(jnp.tanh(anti_ref[...] + sym), 0.0)
    dh_ref[...] = dh
    hn_ref[...] = h_ref[...] + dh


_upd = pl.pallas_call(
    _upd_body,
    grid=(NG,),
    in_specs=[pl.BlockSpec((RB, HID), lambda i: (i, 0)),
              pl.BlockSpec((RB, HID), lambda i: (i, 0)),
              pl.BlockSpec((NC, RB, HID), lambda i: (0, i, 0)),
              pl.BlockSpec((RB, 1), lambda i: (i, 0)),
              pl.BlockSpec((RB, HID), lambda i: (i, 0))],
    out_specs=[pl.BlockSpec((RB, HID), lambda i: (i, 0)),
               pl.BlockSpec((RB, HID), lambda i: (i, 0))],
    out_shape=[jax.ShapeDtypeStruct((N, HID), jnp.float32),
               jax.ShapeDtypeStruct((N, HID), jnp.float32)],
)


def _gate_body(hs_ref, hd_ref, dhs_ref, dhd_ref, rs_ref, rd_ref, ts_ref,
               w1a_ref, w1b_ref, w2d_ref, gd_ref,
               hs_o, hd_o, rs_o, rd_o, ts_o):
    mm = jnp.maximum(
        lax.dot_general(hs_ref[...], w1a_ref[...], _CT1,
                        preferred_element_type=jnp.float32)
        + lax.dot_general(hd_ref[...], w1b_ref[...], _CT1,
                          preferred_element_type=jnp.float32), 0.0)
    d = lax.dot_general(mm, w2d_ref[...], _CT0,
                        preferred_element_type=jnp.float32) + gd_ref[...]
    tau = jnp.where(d >= 0.0, 1.0, 0.0).astype(jnp.float32)
    dhs = dhs_ref[...]
    dhd = dhd_ref[...]
    hs_o[...] = hs_ref[...] + dhs
    hd_o[...] = hd_ref[...] + dhd
    rs_o[...] = rs_ref[...] + tau * dhs
    rd_o[...] = rd_ref[...] + tau * dhd
    ts_o[...] = ts_ref[...] + tau


_gate = pl.pallas_call(
    _gate_body,
    grid=(PG,),
    in_specs=[pl.BlockSpec((PB, HID), lambda i: (i, 0)),
              pl.BlockSpec((PB, HID), lambda i: (i, 0)),
              pl.BlockSpec((PB, HID), lambda i: (i, 0)),
              pl.BlockSpec((PB, HID), lambda i: (i, 0)),
              pl.BlockSpec((PB, HID), lambda i: (i, 0)),
              pl.BlockSpec((PB, HID), lambda i: (i, 0)),
              pl.BlockSpec((PB, 1), lambda i: (i, 0)),
              pl.BlockSpec((2 * HID, HID), lambda i: (0, 0)),
              pl.BlockSpec((2 * HID, HID), lambda i: (0, 0)),
              pl.BlockSpec((2 * HID, 1), lambda i: (0, 0)),
              pl.BlockSpec((PB, 1), lambda i: (i, 0))],
    out_specs=[pl.BlockSpec((PB, HID), lambda i: (i, 0)),
               pl.BlockSpec((PB, HID), lambda i: (i, 0)),
               pl.BlockSpec((PB, HID), lambda i: (i, 0)),
               pl.BlockSpec((PB, HID), lambda i: (i, 0)),
               pl.BlockSpec((PB, 1), lambda i: (i, 0))],
    out_shape=[jax.ShapeDtypeStruct((P, HID), jnp.float32),
               jax.ShapeDtypeStruct((P, HID), jnp.float32),
               jax.ShapeDtypeStruct((P, HID), jnp.float32),
               jax.ShapeDtypeStruct((P, HID), jnp.float32),
               jax.ShapeDtypeStruct((P, 1), jnp.float32)],
)


def _score_body(rs_ref, rd_ref, p1a_ref, p1b_ref, p2_ref, sc_o):
    q = jnp.maximum(
        lax.dot_general(rs_ref[...], p1a_ref[...], _CT1,
                        preferred_element_type=jnp.float32)
        + lax.dot_general(rd_ref[...], p1b_ref[...], _CT1,
                          preferred_element_type=jnp.float32), 0.0)
    sc_o[...] = lax.dot_general(q, p2_ref[...], _CT0,
                                preferred_element_type=jnp.float32)


_score = pl.pallas_call(
    _score_body,
    grid=(PG,),
    in_specs=[pl.BlockSpec((PB, HID), lambda i: (i, 0)),
              pl.BlockSpec((PB, HID), lambda i: (i, 0)),
              pl.BlockSpec((HID, HID), lambda i: (0, 0)),
              pl.BlockSpec((HID, HID), lambda i: (0, 0)),
              pl.BlockSpec((HID, 1), lambda i: (0, 0))],
    out_specs=pl.BlockSpec((PB, 1), lambda i: (i, 0)),
    out_shape=jax.ShapeDtypeStruct((P, 1), jnp.float32),
)


def kernel(x, edge_index, pairs, W_enc, Omega, W_s_raw, W_fc1, W_fc2, W_fv,
           W_p1, W_p2):
    del W_fv
    Omega_as = Omega - Omega.T
    W_s = (W_s_raw + W_s_raw.T) / 2.0
    w1a = W_fc1[:, :HID]
    w1b = W_fc1[:, HID:]
    w2d = (W_fc2[0] - W_fc2[1])[:, None]
    p1a = W_p1[:, :HID]
    p1b = W_p1[:, HID:]
    p2 = W_p2[0][:, None]

    pad = EP - E
    rows_p = jnp.concatenate(
        [edge_index[0], jnp.zeros((pad,), edge_index.dtype)])
    cols_p = jnp.concatenate(
        [edge_index[1], jnp.full((pad,), DUMP, edge_index.dtype)])
    idx_pairs = jnp.concatenate([pairs[:, 0], pairs[:, 1]])

    zeros_nodes = jnp.zeros((NR, HID), jnp.float32)
    zeros_deg = jnp.zeros((NR, DEGW), jnp.float32)
    ones_deg = jnp.ones((C, DEGW), jnp.float32)

    gds = []
    for l in range(L):
        g = jax.random.gumbel(jax.random.fold_in(jax.random.key(42), l),
                              (P, 2), jnp.float32)
        gds.append(g[:, 0:1] - g[:, 1:2])

    edge_scatter, deg_scatter, pair_gather = _sc_kernels()

    H = _encode(x, W_enc)
    degp = deg_scatter(cols_p, ones_deg, zeros_deg)
    dis = _dis_k(degp)

    huv0 = pair_gather(H, idx_pairs)
    hs = huv0[:P]
    hd = huv0[P:]
    rs = hs
    rd = hd
    ts = jnp.zeros((P, 1), jnp.float32)

    for l in range(L):
        anti, Z = _msg(H, Omega_as, W_s, dis)
        pparts = edge_scatter(Z, rows_p, cols_p, zeros_nodes)
        H, dH = _upd(anti, Z, pparts, dis, H)
        dhuv = pair_gather(dH, idx_pairs)
        hs, hd, rs, rd, ts = _gate(hs, hd, dhuv[:P], dhuv[P:], rs, rd, ts,
                                   w1a, w1b, w2d, gds[l])

    scores = _score(rs, rd, p1a, p1b, p2)
    return (scores[:, 0], ts[:, 0])
